# baseline (speedup 1.0000x reference)
import functools

import jax
import jax.numpy as jnp
from jax import lax
from jax.experimental import pallas as pl
from jax.experimental.pallas import tpu as pltpu
from jax.experimental.pallas import tpu_sc as plsc

N = 10000
E = 320000
F_IN = 128
C = 256
CQ = 64
NCLS = 16
NEG = 0.2

NC = 2
NS = 16
L = 16
BATCH = 64
NBAT = 315
EPT = NBAT * BATCH
EPAD = EPT * NS
RPT = N // NS
DP = 10112
RPD = DP // NS


def _split_q(xl, refs):
    for q in range(4):
        refs[q][...] = xl[:, q * CQ:(q + 1) * CQ]


def _pre_body(x_ref, w_ref, acat_ref, x0_ref, x1_ref, x2_ref, x3_ref, al_ref):
    xl = jnp.dot(x_ref[...], w_ref[...], preferred_element_type=jnp.float32)
    _split_q(xl, (x0_ref, x1_ref, x2_ref, x3_ref))
    al_ref[...] = jnp.dot(xl, acat_ref[...], preferred_element_type=jnp.float32)


def _q_outs():
    return tuple(jax.ShapeDtypeStruct((N, CQ), jnp.float32) for _ in range(4))


def _q_specs(n=4):
    return tuple(pl.BlockSpec((2000, CQ), lambda i: (i, 0)) for _ in range(n))


def _pre_call(x, w, acat):
    f = x.shape[1]
    return pl.pallas_call(
        _pre_body,
        out_shape=_q_outs() + (jax.ShapeDtypeStruct((N, 2), jnp.float32),),
        grid=(5,),
        in_specs=[
            pl.BlockSpec((2000, f), lambda i: (i, 0)),
            pl.BlockSpec((f, C), lambda i: (0, 0)),
            pl.BlockSpec((C, 2), lambda i: (0, 0)),
        ],
        out_specs=_q_specs() + (pl.BlockSpec((2000, 2), lambda i: (i, 0)),),
    )(x, w, acat)


def _epilogue(accs, den, al, xqs, b):
    als = al[:, 0:1]
    ald = al[:, 1:2]
    e_self = als + ald
    e_self = jnp.where(e_self >= 0.0, e_self, NEG * e_self)
    ex_self = jnp.exp(e_self)
    xl = jnp.concatenate(xqs, axis=1)
    num = jnp.concatenate(accs, axis=1) + ex_self * xl
    h = num / (den + ex_self + 1e-16)
    return jnp.maximum(h + b, 0.0)


def _mid_body(a0, a1, a2, a3, den_ref, al_ref, p0, p1, p2, p3, b_ref,
              w_ref, acat_ref, x0_ref, x1_ref, x2_ref, x3_ref, al2_ref):
    h = _epilogue((a0[...], a1[...], a2[...], a3[...]), den_ref[...], al_ref[...],
                  (p0[...], p1[...], p2[...], p3[...]), b_ref[...])
    xl = jnp.dot(h, w_ref[...], preferred_element_type=jnp.float32)
    _split_q(xl, (x0_ref, x1_ref, x2_ref, x3_ref))
    al2_ref[...] = jnp.dot(xl, acat_ref[...], preferred_element_type=jnp.float32)


def _mid_call(accs, den, al, xqs, b, w, acat):
    return pl.pallas_call(
        _mid_body,
        out_shape=_q_outs() + (jax.ShapeDtypeStruct((N, 2), jnp.float32),),
        grid=(5,),
        in_specs=[
            *_q_specs(),
            pl.BlockSpec((2000, 1), lambda i: (i, 0)),
            pl.BlockSpec((2000, 2), lambda i: (i, 0)),
            *_q_specs(),
            pl.BlockSpec((1, C), lambda i: (0, 0)),
            pl.BlockSpec((C, C), lambda i: (0, 0)),
            pl.BlockSpec((C, 2), lambda i: (0, 0)),
        ],
        out_specs=_q_specs() + (pl.BlockSpec((2000, 2), lambda i: (i, 0)),),
    )(*accs, den, al, *xqs, b, w, acat)


def _fin_body(a0, a1, a2, a3, den_ref, al_ref, p0, p1, p2, p3, b_ref,
              mw1_ref, mb1_ref, mw2_ref, mb2_ref, o_ref):
    h = _epilogue((a0[...], a1[...], a2[...], a3[...]), den_ref[...], al_ref[...],
                  (p0[...], p1[...], p2[...], p3[...]), b_ref[...])
    t = jnp.dot(h, mw1_ref[...], preferred_element_type=jnp.float32) + mb1_ref[...]
    t = jnp.maximum(t, 0.0)
    o = jnp.dot(t, mw2_ref[...], preferred_element_type=jnp.float32) + mb2_ref[...]
    o_ref[...] = jax.nn.sigmoid(o)


def _fin_call(accs, den, al, xqs, b, mw1, mb1, mw2, mb2):
    return pl.pallas_call(
        _fin_body,
        out_shape=jax.ShapeDtypeStruct((N, NCLS), jnp.float32),
        grid=(5,),
        in_specs=[
            *_q_specs(),
            pl.BlockSpec((2000, 1), lambda i: (i, 0)),
            pl.BlockSpec((2000, 2), lambda i: (i, 0)),
            *_q_specs(),
            pl.BlockSpec((1, C), lambda i: (0, 0)),
            pl.BlockSpec((C, C), lambda i: (0, 0)),
            pl.BlockSpec((1, C), lambda i: (0, 0)),
            pl.BlockSpec((C, NCLS), lambda i: (0, 0)),
            pl.BlockSpec((1, NCLS), lambda i: (0, 0)),
        ],
        out_specs=pl.BlockSpec((2000, NCLS), lambda i: (i, 0)),
    )(*accs, den, al, *xqs, b, mw1, mb1, mw2, mb2)


_sc_mesh = plsc.VectorSubcoreMesh(core_axis_name="c", subcore_axis_name="s")


@functools.partial(
    pl.kernel,
    out_type=(
        tuple(jax.ShapeDtypeStruct((N, CQ), jnp.float32) for _ in range(4))
        + (jax.ShapeDtypeStruct((DP,), jnp.float32),)
    ),
    mesh=_sc_mesh,
    compiler_params=pltpu.CompilerParams(needs_layout_passes=False,
                                         use_tc_tiling_on_sc=False),
    scratch_types=[
        pltpu.VMEM((2 * N,), jnp.float32),
        pltpu.VMEM((EPT,), jnp.int32),
        pltpu.VMEM((EPT,), jnp.int32),
        pltpu.VMEM((EPT,), jnp.float32),
        pltpu.VMEM((BATCH,), jnp.int32),
        pltpu.VMEM((BATCH,), jnp.int32),
        pltpu.VMEM((BATCH,), jnp.int32),
        pltpu.VMEM((BATCH,), jnp.int32),
        pltpu.VMEM((BATCH, CQ), jnp.float32),
        pltpu.VMEM((BATCH, CQ), jnp.float32),
        pltpu.VMEM_SHARED((N, CQ), jnp.float32),
        pltpu.VMEM_SHARED((DP,), jnp.float32),
        pltpu.SemaphoreType.DMA,
        pltpu.SemaphoreType.DMA,
    ],
)
def _edge_kernel(src_hbm, dst_hbm, alf_hbm, x0_hbm, x1_hbm, x2_hbm, x3_hbm,
                 zacc_hbm, zden_hbm,
                 q0_out, q1_out, q2_out, q3_out, den_out,
                 alf_v, src_v, dst_v, ex_v, sidx0_v, didx0_v, sidx1_v, didx1_v,
                 rows0_v, rows1_v, acc_sh, den_sh, sem0, sem1):
    c = lax.axis_index("c")
    s = lax.axis_index("s")

    pltpu.sync_copy(alf_hbm, alf_v)
    ebase = s * EPT
    pltpu.sync_copy(src_hbm.at[pl.ds(ebase, EPT)], src_v)
    pltpu.sync_copy(dst_hbm.at[pl.ds(ebase, EPT)], dst_v)

    rsl = pl.ds(s * RPT, RPT)
    dsl = pl.ds(s * RPD, RPD)

    @pl.when(c == 0)
    def _():
        pltpu.sync_copy(zden_hbm.at[dsl], den_sh.at[dsl])

    lanes = lax.iota(jnp.int32, L)

    def pass_a(i, carry):
        sl = pl.ds(i * L, L)
        isrc = src_v[sl]
        idst = dst_v[sl]
        a = (plsc.load_gather(alf_v, [isrc * 2])
             + plsc.load_gather(alf_v, [idst * 2 + 1]))
        a = jnp.where(a >= 0.0, a, NEG * a)
        gidx = ebase + i * L + lanes
        ex_v[sl] = jnp.where(gidx < E, jnp.exp(a), 0.0)
        return carry

    lax.fori_loop(0, EPT // L, pass_a, 0)

    def tbl(p):
        def pick(r01, r23):
            def f(sidx_v, rows_v, op):
                @pl.when(c == 0)
                def _():
                    op(pltpu.make_async_copy(r01.at[sidx_v], rows_v, sem0
                                             if rows_v is rows0_v else sem1))

                @pl.when(c == 1)
                def _():
                    op(pltpu.make_async_copy(r23.at[sidx_v], rows_v, sem0
                                             if rows_v is rows0_v else sem1))
            return f
        return pick(x0_hbm if p == 0 else x1_hbm, x2_hbm if p == 0 else x3_hbm)

    def gstart(gfn, g, sidx_v, didx_v, rows_v):
        off = pl.multiple_of(g * BATCH, BATCH)

        def cp(j, carry):
            jl = pl.ds(j * L, L)
            sidx_v[jl] = src_v[pl.ds(off + j * L, L)]
            didx_v[jl] = dst_v[pl.ds(off + j * L, L)]
            return carry

        lax.fori_loop(0, BATCH // L, cp, 0)
        gfn(sidx_v, rows_v, lambda d: d.start())

    def gwait(gfn, sidx_v, rows_v):
        gfn(sidx_v, rows_v, lambda d: d.wait())

    def process(g, didx_v, rows_v, p):
        off = pl.multiple_of(g * BATCH, BATCH)

        def scale(gg, carry):
            exvec = ex_v[pl.ds(off + gg * L, L)]
            for lane in range(L):
                t = exvec[lane]
                for j in range(CQ // L):
                    fs = pl.ds(j * L, L)
                    rows_v[gg * L + lane, fs] = rows_v[gg * L + lane, fs] * t
            return carry

        lax.fori_loop(0, BATCH // L, scale, 0)
        pltpu.sync_copy(rows_v, acc_sh.at[didx_v], add=True)

        if p == 0:
            @pl.when(c == 0)
            def _():
                pltpu.sync_copy(ex_v.at[pl.ds(off, BATCH)],
                                den_sh.at[didx_v], add=True)

    for p in range(2):
        gfn = tbl(p)
        pltpu.sync_copy(zacc_hbm.at[rsl], acc_sh.at[rsl])
        plsc.subcore_barrier()

        gstart(gfn, 0, sidx0_v, didx0_v, rows0_v)

        def pass_b(t, carry, gfn=gfn, p=p):
            g0 = t * 2
            gwait(gfn, sidx0_v, rows0_v)
            gstart(gfn, g0 + 1, sidx1_v, didx1_v, rows1_v)
            process(g0, didx0_v, rows0_v, p)
            gwait(gfn, sidx1_v, rows1_v)
            gstart(gfn, g0 + 2, sidx0_v, didx0_v, rows0_v)
            process(g0 + 1, didx1_v, rows1_v, p)
            return carry

        lax.fori_loop(0, (NBAT - 1) // 2, pass_b, 0)
        gwait(gfn, sidx0_v, rows0_v)
        process(NBAT - 1, didx0_v, rows0_v, p)

        plsc.subcore_barrier()

        @pl.when(c == 0)
        def _():
            if p == 0:
                pltpu.sync_copy(acc_sh.at[rsl], q0_out.at[rsl])
                pltpu.sync_copy(den_sh.at[dsl], den_out.at[dsl])
            else:
                pltpu.sync_copy(acc_sh.at[rsl], q1_out.at[rsl])

        @pl.when(c == 1)
        def _():
            if p == 0:
                pltpu.sync_copy(acc_sh.at[rsl], q2_out.at[rsl])
            else:
                pltpu.sync_copy(acc_sh.at[rsl], q3_out.at[rsl])


def kernel(x, edge_index, W1, as1, ad1, b1, W2, as2, ad2, b2, mw1, mb1, mw2, mb2):
    pad = jnp.zeros((EPAD - E,), edge_index.dtype)
    src = jnp.concatenate([edge_index[0], pad])
    dst = jnp.concatenate([edge_index[1], pad])
    acat1 = jnp.concatenate([as1, ad1], axis=0).T
    acat2 = jnp.concatenate([as2, ad2], axis=0).T
    zacc = jnp.zeros((N, CQ), jnp.float32)
    zden = jnp.zeros((DP,), jnp.float32)

    *xq1, al1 = _pre_call(x, W1, acat1)
    *acc1, den1 = _edge_kernel(src, dst, al1.reshape(2 * N), *xq1, zacc, zden)
    *xq2, al2 = _mid_call(tuple(acc1), den1.reshape(DP, 1), al1, tuple(xq1),
                          b1.reshape(1, C), W2, acat2)
    *acc2, den2 = _edge_kernel(src, dst, al2.reshape(2 * N), *xq2, zacc, zden)
    out = _fin_call(tuple(acc2), den2.reshape(DP, 1), al2, tuple(xq2),
                    b2.reshape(1, C), mw1, mb1.reshape(1, C), mw2,
                    mb2.reshape(1, NCLS))
    return out
